# baseline (speedup 1.0000x reference)
import jax
import jax.numpy as jnp
from jax import lax
from jax.experimental import pallas as pl
from jax.experimental.pallas import tpu as pltpu
from jax.experimental.pallas import tpu_sc as plsc

N = 10000
E = 160000
D = 128
L = 4

NC = 2
NS = 16
NW = NC * NS

HS = (25600, 108800, 25600)

GCW = 80
SCW = 40
NPAD = 10240
NPC = NPAD // NS
ZCW = 128

BE = 1280
BN = 2000


_R = 4


def _make_gather(nch):
    def body(t_hbm, idx_hbm, out_hbm, idx_v, *rest):
        bufs = rest[:_R]
        sems = rest[_R:2 * _R]
        osems = rest[2 * _R:3 * _R]
        wid = lax.axis_index("c") * NS + lax.axis_index("s")
        pltpu.sync_copy(idx_hbm.at[wid], idx_v)
        base = wid * (nch * GCW)

        for k in range(_R - 1):
            pltpu.async_copy(t_hbm.at[idx_v.at[k]], bufs[k], sems[k])

        def chunk(j, b, static):
            pltpu.make_async_copy(t_hbm.at[pl.ds(0, GCW)], bufs[b],
                                  sems[b]).wait()
            pb = (b - 1) % _R

            def drain():
                pltpu.make_async_copy(bufs[pb], out_hbm.at[pl.ds(base, GCW)],
                                      osems[pb]).wait()

            def prefetch():
                pltpu.async_copy(t_hbm.at[idx_v.at[j + _R - 1]], bufs[pb],
                                 sems[pb])

            if static:
                if j >= 1:
                    drain()
                if j + _R - 1 < nch:
                    prefetch()
            else:
                pl.when(j >= 1)(drain)
                pl.when(j + _R - 1 < nch)(prefetch)
            pltpu.async_copy(bufs[b], out_hbm.at[pl.ds(base + j * GCW, GCW)],
                             osems[b])

        def step(jj, carry):
            for b in range(_R):
                chunk(jj * _R + b, b, False)
            return carry

        lax.fori_loop(0, nch // _R, step, 0)
        for k in range(nch % _R):
            j = (nch // _R) * _R + k
            chunk(j, j % _R, True)
        lb = (nch - 1) % _R
        pltpu.make_async_copy(bufs[lb], out_hbm.at[pl.ds(base, GCW)],
                              osems[lb]).wait()

    def run(tflat, idx3):
        return pl.kernel(
            body,
            out_type=jax.ShapeDtypeStruct((NW * nch * GCW, D), jnp.float32),
            mesh=plsc.VectorSubcoreMesh(core_axis_name="c",
                                        subcore_axis_name="s"),
            scratch_types=(
                [pltpu.VMEM((nch, GCW), jnp.int32)]
                + [pltpu.VMEM((GCW, D), jnp.float32)] * _R
                + [pltpu.SemaphoreType.DMA] * (2 * _R)
            ),
        )(tflat, idx3)

    return run


def _make_scatter(nch):
    def body(e_hbm, idx_hbm, out_hbm, idx_v, zbuf, shared, *rest):
        bufs = rest[:_R]
        sems = rest[_R:2 * _R]
        asems = rest[2 * _R:3 * _R]
        cid = lax.axis_index("c")
        sid = lax.axis_index("s")
        wid = cid * NS + sid
        base = wid * (nch * SCW)
        pltpu.sync_copy(idx_hbm.at[wid], idx_v)
        for k in range(_R - 1):
            pltpu.async_copy(e_hbm.at[pl.ds(base + k * SCW, SCW)], bufs[k],
                             sems[k])

        def zrow(i, carry):
            r = i // (D // 16)
            c = (i % (D // 16)) * 16
            zbuf[r, pl.ds(c, 16)] = jnp.zeros((16,), jnp.float32)
            return carry

        lax.fori_loop(0, ZCW * (D // 16), zrow, 0)
        for k in range(NPC // ZCW):
            pltpu.sync_copy(zbuf, shared.at[pl.ds(sid * NPC + k * ZCW, ZCW)])
        plsc.subcore_barrier()

        def chunk(j, b, static):
            pltpu.make_async_copy(e_hbm.at[pl.ds(0, SCW)], bufs[b],
                                  sems[b]).wait()
            pltpu.async_copy(bufs[b], shared.at[idx_v.at[j]], asems[b],
                             add=True)
            pb = (b - 1) % _R

            def drain():
                pltpu.make_async_copy(bufs[pb], shared.at[pl.ds(0, SCW)],
                                      asems[pb]).wait()

            def prefetch():
                pltpu.async_copy(
                    e_hbm.at[pl.ds(base + (j + _R - 1) * SCW, SCW)],
                    bufs[pb], sems[pb])

            if static:
                if j >= 1:
                    drain()
                if j + _R - 1 < nch:
                    prefetch()
            else:
                pl.when(j >= 1)(drain)
                pl.when(j + _R - 1 < nch)(prefetch)

        def step(jj, carry):
            for b in range(_R):
                chunk(jj * _R + b, b, False)
            return carry

        lax.fori_loop(0, nch // _R, step, 0)
        for k in range(nch % _R):
            j = (nch // _R) * _R + k
            chunk(j, j % _R, True)
        lb = (nch - 1) % _R
        pltpu.make_async_copy(bufs[lb], shared.at[pl.ds(0, SCW)],
                              asems[lb]).wait()
        plsc.subcore_barrier()
        pltpu.sync_copy(shared.at[pl.ds(sid * NPC, NPC)],
                        out_hbm.at[cid, pl.ds(sid * NPC, NPC)])

    def run(e, idx3):
        return pl.kernel(
            body,
            out_type=jax.ShapeDtypeStruct((NC, NPAD, D), jnp.float32),
            mesh=plsc.VectorSubcoreMesh(core_axis_name="c",
                                        subcore_axis_name="s"),
            scratch_types=(
                [pltpu.VMEM((nch, SCW), jnp.int32),
                 pltpu.VMEM((ZCW, D), jnp.float32),
                 pltpu.VMEM_SHARED((NPAD, D), jnp.float32)]
                + [pltpu.VMEM((SCW, D), jnp.float32)] * _R
                + [pltpu.SemaphoreType.DMA] * (2 * _R)
            ),
        )(e, idx3)

    return run


_GATHERS = tuple(_make_gather(2 * h // NW // GCW) for h in HS)
_SCATTERS = tuple(_make_scatter(h // NW // SCW) for h in HS)


def _mm(a, w):
    return jnp.dot(a.astype(jnp.bfloat16), w.astype(jnp.bfloat16),
                   preferred_element_type=jnp.float32)


def _layer_norm(h, g, b):
    mu = jnp.mean(h, axis=-1, keepdims=True)
    var = jnp.mean((h - mu) ** 2, axis=-1, keepdims=True)
    return (h - mu) * lax.rsqrt(var + 1e-5) * g + b


def _pre_body(x_ref, wsd_ref, t_ref):
    x = x_ref[...]
    t_ref[0] = _mm(x, wsd_ref[0])
    t_ref[1] = _mm(x, wsd_ref[1])


def _preproj(x, wsd):
    nb = N // BN
    return pl.pallas_call(
        _pre_body,
        grid=(nb,),
        in_specs=[
            pl.BlockSpec((BN, D), lambda i: (i, 0)),
            pl.BlockSpec((2, D, D), lambda i: (0, 0, 0)),
        ],
        out_specs=pl.BlockSpec((2, BN, D), lambda i: (0, i, 0)),
        out_shape=jax.ShapeDtypeStruct((2, N, D), jnp.float32),
    )(x, wsd)


def _edge_body(e_ref, gs_ref, gd_ref, w1_ref, w2_ref, w3_ref,
               b1_ref, b2_ref, b3_ref, g_ref, bt_ref, o_ref):
    e = e_ref[...]
    h = _mm(e, w1_ref[...])
    h = h + gs_ref[...] + gd_ref[...]
    h = jnp.maximum(h + b1_ref[...], 0.0)
    h = jnp.maximum(_mm(h, w2_ref[...]) + b2_ref[...], 0.0)
    h = _mm(h, w3_ref[...]) + b3_ref[...]
    o_ref[...] = _layer_norm(h, g_ref[...], bt_ref[...]) + e


def _edge_mlp(e, G, w1e, w2, w3, b1, b2, b3, g, bt):
    nr = e.shape[0]
    nb = nr // BE
    vspec = pl.BlockSpec((1, D), lambda i: (0, 0))
    wspec = pl.BlockSpec((D, D), lambda i: (0, 0))
    return pl.pallas_call(
        _edge_body,
        grid=(nb,),
        in_specs=[
            pl.BlockSpec((BE, D), lambda i: (i, 0)),
            pl.BlockSpec((BE, D), lambda i: (i, 0)),
            pl.BlockSpec((BE, D), lambda i: (i + nb, 0)),
            wspec, wspec, wspec, vspec, vspec, vspec, vspec, vspec,
        ],
        out_specs=pl.BlockSpec((BE, D), lambda i: (i, 0)),
        out_shape=jax.ShapeDtypeStruct((nr, D), jnp.float32),
    )(e, G, G, w1e, w2, w3, b1, b2, b3, g, bt)


def _node_body(*refs):
    aggs = refs[:len(HS)]
    (x_ref, w1a_ref, w1x_ref, w2_ref, w3_ref,
     b1_ref, b2_ref, b3_ref, g_ref, bt_ref, *rest) = refs[len(HS):]
    x = x_ref[...]
    a = aggs[0][0] + aggs[0][1]
    for ar in aggs[1:]:
        a = a + ar[0] + ar[1]
    h = _mm(a, w1a_ref[...]) + _mm(x, w1x_ref[...])
    h = jnp.maximum(h + b1_ref[...], 0.0)
    h = jnp.maximum(_mm(h, w2_ref[...]) + b2_ref[...], 0.0)
    h = _mm(h, w3_ref[...]) + b3_ref[...]
    xn = _layer_norm(h, g_ref[...], bt_ref[...]) + x
    if len(rest) == 1:
        rest[0][...] = xn
    else:
        wsd_ref, xo_ref, t_ref = rest
        xo_ref[...] = xn
        t_ref[0] = _mm(xn, wsd_ref[0])
        t_ref[1] = _mm(xn, wsd_ref[1])


def _node_mlp(aggs, x, w1a, w1x, w2, w3, b1, b2, b3, g, bt, wsd_next):
    nb = N // BN
    vspec = pl.BlockSpec((1, D), lambda i: (0, 0))
    wspec = pl.BlockSpec((D, D), lambda i: (0, 0))
    aspec = pl.BlockSpec((NC, BN, D), lambda i: (0, i, 0))
    in_specs = [aspec] * len(HS) + [
        pl.BlockSpec((BN, D), lambda i: (i, 0)),
        wspec, wspec, wspec, wspec, vspec, vspec, vspec, vspec, vspec,
    ]
    xspec = pl.BlockSpec((BN, D), lambda i: (i, 0))
    xshape = jax.ShapeDtypeStruct((N, D), jnp.float32)
    args = list(aggs) + [x, w1a, w1x, w2, w3, b1, b2, b3, g, bt]
    if wsd_next is None:
        return pl.pallas_call(
            _node_body, grid=(nb,), in_specs=in_specs,
            out_specs=xspec, out_shape=xshape,
        )(*args)
    in_specs.append(pl.BlockSpec((2, D, D), lambda i: (0, 0, 0)))
    return pl.pallas_call(
        _node_body, grid=(nb,), in_specs=in_specs,
        out_specs=(xspec, pl.BlockSpec((2, BN, D), lambda i: (0, i, 0))),
        out_shape=(xshape, jax.ShapeDtypeStruct((2, N, D), jnp.float32)),
    )(*args, wsd_next)


def kernel(node_features, edge_features, edge_index, params):
    src = edge_index[0]
    dst = edge_index[1]
    e_parts = []
    idx_g = []
    idx_s = []
    off = 0
    for h in HS:
        e_parts.append(lax.dynamic_slice(edge_features, (off, 0), (h, D)))
        s_h = lax.dynamic_slice(src, (off,), (h,))
        d_h = lax.dynamic_slice(dst, (off,), (h,))
        idx_g.append(jnp.concatenate([s_h, d_h + N])
                     .reshape(NW, 2 * h // NW // GCW, GCW))
        idx_s.append(d_h.reshape(NW, h // NW // SCW, SCW))
        off += h
    e_parts = tuple(e_parts)

    def wsd(l):
        w1 = params['e%d_W1' % l]
        return jnp.stack([w1[D:2 * D], w1[2 * D:3 * D]])

    def v(pre, s):
        return params[pre + s].reshape(1, D)

    x = node_features
    T = _preproj(x, wsd(0))
    for l in range(L):
        ep = 'e%d' % l
        np_ = 'n%d' % l
        tflat = T.reshape(2 * N, D)
        new_parts = []
        aggs = []
        for h in range(len(HS)):
            G = _GATHERS[h](tflat, idx_g[h])
            eh = _edge_mlp(e_parts[h], G, params[ep + '_W1'][:D],
                           params[ep + '_W2'], params[ep + '_W3'],
                           v(ep, '_b1'), v(ep, '_b2'), v(ep, '_b3'),
                           v(ep, '_g'), v(ep, '_beta'))
            new_parts.append(eh)
            aggs.append(_SCATTERS[h](eh, idx_s[h]))
        e_parts = tuple(new_parts)
        out = _node_mlp(aggs, x, params[np_ + '_W1'][:D],
                        params[np_ + '_W1'][D:], params[np_ + '_W2'],
                        params[np_ + '_W3'], v(np_, '_b1'), v(np_, '_b2'),
                        v(np_, '_b3'), v(np_, '_g'), v(np_, '_beta'),
                        wsd(l + 1) if l < L - 1 else None)
        if l < L - 1:
            x, T = out
        else:
            x = out
    return x


# speedup vs baseline: 1.0299x; 1.0299x over previous
import jax
import jax.numpy as jnp
from jax import lax
from jax.experimental import pallas as pl
from jax.experimental.pallas import tpu as pltpu
from jax.experimental.pallas import tpu_sc as plsc

N = 10000
E = 160000
D = 128
L = 4

NC = 2
NS = 16
NW = NC * NS

HS = (33280, 93440, 33280)

GCW = 80
SCW = 40
NPAD = 10240
NPC = NPAD // NS
ZCW = 128

BE = 1280
BN = 2000


_R = 4


def _make_gather(nch):
    def body(t_hbm, idx_hbm, out_hbm, idx_v, *rest):
        bufs = rest[:_R]
        sems = rest[_R:2 * _R]
        osems = rest[2 * _R:3 * _R]
        wid = lax.axis_index("c") * NS + lax.axis_index("s")
        pltpu.sync_copy(idx_hbm.at[wid], idx_v)
        base = wid * (nch * GCW)

        for k in range(_R - 1):
            pltpu.async_copy(t_hbm.at[idx_v.at[k]], bufs[k], sems[k])

        def chunk(j, b, static):
            pltpu.make_async_copy(t_hbm.at[pl.ds(0, GCW)], bufs[b],
                                  sems[b]).wait()
            pb = (b - 1) % _R

            def drain():
                pltpu.make_async_copy(bufs[pb], out_hbm.at[pl.ds(base, GCW)],
                                      osems[pb]).wait()

            def prefetch():
                pltpu.async_copy(t_hbm.at[idx_v.at[j + _R - 1]], bufs[pb],
                                 sems[pb])

            if static:
                if j >= 1:
                    drain()
                if j + _R - 1 < nch:
                    prefetch()
            else:
                pl.when(j >= 1)(drain)
                pl.when(j + _R - 1 < nch)(prefetch)
            pltpu.async_copy(bufs[b], out_hbm.at[pl.ds(base + j * GCW, GCW)],
                             osems[b])

        def step(jj, carry):
            for b in range(_R):
                chunk(jj * _R + b, b, False)
            return carry

        lax.fori_loop(0, nch // _R, step, 0)
        for k in range(nch % _R):
            j = (nch // _R) * _R + k
            chunk(j, j % _R, True)
        lb = (nch - 1) % _R
        pltpu.make_async_copy(bufs[lb], out_hbm.at[pl.ds(base, GCW)],
                              osems[lb]).wait()

    def run(tflat, idx3):
        return pl.kernel(
            body,
            out_type=jax.ShapeDtypeStruct((NW * nch * GCW, D), jnp.float32),
            mesh=plsc.VectorSubcoreMesh(core_axis_name="c",
                                        subcore_axis_name="s"),
            scratch_types=(
                [pltpu.VMEM((nch, GCW), jnp.int32)]
                + [pltpu.VMEM((GCW, D), jnp.float32)] * _R
                + [pltpu.SemaphoreType.DMA] * (2 * _R)
            ),
        )(tflat, idx3)

    return run


def _make_scatter(nch):
    def body(e_hbm, idx_hbm, out_hbm, idx_v, zbuf, shared, *rest):
        bufs = rest[:_R]
        sems = rest[_R:2 * _R]
        asems = rest[2 * _R:3 * _R]
        cid = lax.axis_index("c")
        sid = lax.axis_index("s")
        wid = cid * NS + sid
        base = wid * (nch * SCW)
        pltpu.sync_copy(idx_hbm.at[wid], idx_v)
        for k in range(_R - 1):
            pltpu.async_copy(e_hbm.at[pl.ds(base + k * SCW, SCW)], bufs[k],
                             sems[k])

        def zrow(i, carry):
            r = i // (D // 16)
            c = (i % (D // 16)) * 16
            zbuf[r, pl.ds(c, 16)] = jnp.zeros((16,), jnp.float32)
            return carry

        lax.fori_loop(0, ZCW * (D // 16), zrow, 0)
        for k in range(NPC // ZCW):
            pltpu.sync_copy(zbuf, shared.at[pl.ds(sid * NPC + k * ZCW, ZCW)])
        plsc.subcore_barrier()

        def chunk(j, b, static):
            pltpu.make_async_copy(e_hbm.at[pl.ds(0, SCW)], bufs[b],
                                  sems[b]).wait()
            pltpu.async_copy(bufs[b], shared.at[idx_v.at[j]], asems[b],
                             add=True)
            pb = (b - 1) % _R

            def drain():
                pltpu.make_async_copy(bufs[pb], shared.at[pl.ds(0, SCW)],
                                      asems[pb]).wait()

            def prefetch():
                pltpu.async_copy(
                    e_hbm.at[pl.ds(base + (j + _R - 1) * SCW, SCW)],
                    bufs[pb], sems[pb])

            if static:
                if j >= 1:
                    drain()
                if j + _R - 1 < nch:
                    prefetch()
            else:
                pl.when(j >= 1)(drain)
                pl.when(j + _R - 1 < nch)(prefetch)

        def step(jj, carry):
            for b in range(_R):
                chunk(jj * _R + b, b, False)
            return carry

        lax.fori_loop(0, nch // _R, step, 0)
        for k in range(nch % _R):
            j = (nch // _R) * _R + k
            chunk(j, j % _R, True)
        lb = (nch - 1) % _R
        pltpu.make_async_copy(bufs[lb], shared.at[pl.ds(0, SCW)],
                              asems[lb]).wait()
        plsc.subcore_barrier()
        pltpu.sync_copy(shared.at[pl.ds(sid * NPC, NPC)],
                        out_hbm.at[cid, pl.ds(sid * NPC, NPC)])

    def run(e, idx3):
        return pl.kernel(
            body,
            out_type=jax.ShapeDtypeStruct((NC, NPAD, D), jnp.float32),
            mesh=plsc.VectorSubcoreMesh(core_axis_name="c",
                                        subcore_axis_name="s"),
            scratch_types=(
                [pltpu.VMEM((nch, SCW), jnp.int32),
                 pltpu.VMEM((ZCW, D), jnp.float32),
                 pltpu.VMEM_SHARED((NPAD, D), jnp.float32)]
                + [pltpu.VMEM((SCW, D), jnp.float32)] * _R
                + [pltpu.SemaphoreType.DMA] * (2 * _R)
            ),
        )(e, idx3)

    return run


_GATHERS = tuple(_make_gather(2 * h // NW // GCW) for h in HS)
_SCATTERS = tuple(_make_scatter(h // NW // SCW) for h in HS)


def _mm(a, w):
    return jnp.dot(a.astype(jnp.bfloat16), w.astype(jnp.bfloat16),
                   preferred_element_type=jnp.float32)


def _layer_norm(h, g, b):
    mu = jnp.mean(h, axis=-1, keepdims=True)
    var = jnp.mean((h - mu) ** 2, axis=-1, keepdims=True)
    return (h - mu) * lax.rsqrt(var + 1e-5) * g + b


def _pre_body(x_ref, wsd_ref, t_ref):
    x = x_ref[...]
    t_ref[0] = _mm(x, wsd_ref[0])
    t_ref[1] = _mm(x, wsd_ref[1])


def _preproj(x, wsd):
    nb = N // BN
    return pl.pallas_call(
        _pre_body,
        grid=(nb,),
        in_specs=[
            pl.BlockSpec((BN, D), lambda i: (i, 0)),
            pl.BlockSpec((2, D, D), lambda i: (0, 0, 0)),
        ],
        out_specs=pl.BlockSpec((2, BN, D), lambda i: (0, i, 0)),
        out_shape=jax.ShapeDtypeStruct((2, N, D), jnp.float32),
    )(x, wsd)


def _edge_body(e_ref, gs_ref, gd_ref, w1_ref, w2_ref, w3_ref,
               b1_ref, b2_ref, b3_ref, g_ref, bt_ref, o_ref):
    e = e_ref[...]
    h = _mm(e, w1_ref[...])
    h = h + gs_ref[...] + gd_ref[...]
    h = jnp.maximum(h + b1_ref[...], 0.0)
    h = jnp.maximum(_mm(h, w2_ref[...]) + b2_ref[...], 0.0)
    h = _mm(h, w3_ref[...]) + b3_ref[...]
    o_ref[...] = _layer_norm(h, g_ref[...], bt_ref[...]) + e


def _edge_mlp(e, G, w1e, w2, w3, b1, b2, b3, g, bt):
    nr = e.shape[0]
    nb = nr // BE
    vspec = pl.BlockSpec((1, D), lambda i: (0, 0))
    wspec = pl.BlockSpec((D, D), lambda i: (0, 0))
    return pl.pallas_call(
        _edge_body,
        grid=(nb,),
        in_specs=[
            pl.BlockSpec((BE, D), lambda i: (i, 0)),
            pl.BlockSpec((BE, D), lambda i: (i, 0)),
            pl.BlockSpec((BE, D), lambda i: (i + nb, 0)),
            wspec, wspec, wspec, vspec, vspec, vspec, vspec, vspec,
        ],
        out_specs=pl.BlockSpec((BE, D), lambda i: (i, 0)),
        out_shape=jax.ShapeDtypeStruct((nr, D), jnp.float32),
    )(e, G, G, w1e, w2, w3, b1, b2, b3, g, bt)


def _node_body(*refs):
    aggs = refs[:len(HS)]
    (x_ref, w1a_ref, w1x_ref, w2_ref, w3_ref,
     b1_ref, b2_ref, b3_ref, g_ref, bt_ref, *rest) = refs[len(HS):]
    x = x_ref[...]
    a = aggs[0][0] + aggs[0][1]
    for ar in aggs[1:]:
        a = a + ar[0] + ar[1]
    h = _mm(a, w1a_ref[...]) + _mm(x, w1x_ref[...])
    h = jnp.maximum(h + b1_ref[...], 0.0)
    h = jnp.maximum(_mm(h, w2_ref[...]) + b2_ref[...], 0.0)
    h = _mm(h, w3_ref[...]) + b3_ref[...]
    xn = _layer_norm(h, g_ref[...], bt_ref[...]) + x
    if len(rest) == 1:
        rest[0][...] = xn
    else:
        wsd_ref, xo_ref, t_ref = rest
        xo_ref[...] = xn
        t_ref[0] = _mm(xn, wsd_ref[0])
        t_ref[1] = _mm(xn, wsd_ref[1])


def _node_mlp(aggs, x, w1a, w1x, w2, w3, b1, b2, b3, g, bt, wsd_next):
    nb = N // BN
    vspec = pl.BlockSpec((1, D), lambda i: (0, 0))
    wspec = pl.BlockSpec((D, D), lambda i: (0, 0))
    aspec = pl.BlockSpec((NC, BN, D), lambda i: (0, i, 0))
    in_specs = [aspec] * len(HS) + [
        pl.BlockSpec((BN, D), lambda i: (i, 0)),
        wspec, wspec, wspec, wspec, vspec, vspec, vspec, vspec, vspec,
    ]
    xspec = pl.BlockSpec((BN, D), lambda i: (i, 0))
    xshape = jax.ShapeDtypeStruct((N, D), jnp.float32)
    args = list(aggs) + [x, w1a, w1x, w2, w3, b1, b2, b3, g, bt]
    if wsd_next is None:
        return pl.pallas_call(
            _node_body, grid=(nb,), in_specs=in_specs,
            out_specs=xspec, out_shape=xshape,
        )(*args)
    in_specs.append(pl.BlockSpec((2, D, D), lambda i: (0, 0, 0)))
    return pl.pallas_call(
        _node_body, grid=(nb,), in_specs=in_specs,
        out_specs=(xspec, pl.BlockSpec((2, BN, D), lambda i: (0, i, 0))),
        out_shape=(xshape, jax.ShapeDtypeStruct((2, N, D), jnp.float32)),
    )(*args, wsd_next)


def kernel(node_features, edge_features, edge_index, params):
    src = edge_index[0]
    dst = edge_index[1]
    e_parts = []
    idx_g = []
    idx_s = []
    off = 0
    for h in HS:
        e_parts.append(lax.dynamic_slice(edge_features, (off, 0), (h, D)))
        s_h = lax.dynamic_slice(src, (off,), (h,))
        d_h = lax.dynamic_slice(dst, (off,), (h,))
        idx_g.append(jnp.concatenate([s_h, d_h + N])
                     .reshape(NW, 2 * h // NW // GCW, GCW))
        idx_s.append(d_h.reshape(NW, h // NW // SCW, SCW))
        off += h
    e_parts = tuple(e_parts)

    def wsd(l):
        w1 = params['e%d_W1' % l]
        return jnp.stack([w1[D:2 * D], w1[2 * D:3 * D]])

    def v(pre, s):
        return params[pre + s].reshape(1, D)

    x = node_features
    T = _preproj(x, wsd(0))
    for l in range(L):
        ep = 'e%d' % l
        np_ = 'n%d' % l
        tflat = T.reshape(2 * N, D)
        new_parts = []
        aggs = []
        for h in range(len(HS)):
            G = _GATHERS[h](tflat, idx_g[h])
            eh = _edge_mlp(e_parts[h], G, params[ep + '_W1'][:D],
                           params[ep + '_W2'], params[ep + '_W3'],
                           v(ep, '_b1'), v(ep, '_b2'), v(ep, '_b3'),
                           v(ep, '_g'), v(ep, '_beta'))
            new_parts.append(eh)
            aggs.append(_SCATTERS[h](eh, idx_s[h]))
        e_parts = tuple(new_parts)
        out = _node_mlp(aggs, x, params[np_ + '_W1'][:D],
                        params[np_ + '_W1'][D:], params[np_ + '_W2'],
                        params[np_ + '_W3'], v(np_, '_b1'), v(np_, '_b2'),
                        v(np_, '_b3'), v(np_, '_g'), v(np_, '_beta'),
                        wsd(l + 1) if l < L - 1 else None)
        if l < L - 1:
            x, T = out
        else:
            x = out
    return x


# speedup vs baseline: 1.0540x; 1.0235x over previous
import jax
import jax.numpy as jnp
from jax import lax
from jax.experimental import pallas as pl
from jax.experimental.pallas import tpu as pltpu
from jax.experimental.pallas import tpu_sc as plsc

N = 10000
E = 160000
D = 128
L = 4

NC = 2
NS = 16
NW = NC * NS

HS = (39680, 80640, 39680)

GCW = 80
SCW = 40
NPAD = 10240
NPC = NPAD // NS
ZCW = 128

BE = 1280
BN = 2000


_R = 4


def _make_gather(nch):
    def body(t_hbm, idx_hbm, out_hbm, idx_v, *rest):
        bufs = rest[:_R]
        sems = rest[_R:2 * _R]
        osems = rest[2 * _R:3 * _R]
        wid = lax.axis_index("c") * NS + lax.axis_index("s")
        pltpu.sync_copy(idx_hbm.at[wid], idx_v)
        base = wid * (nch * GCW)

        for k in range(_R - 1):
            pltpu.async_copy(t_hbm.at[idx_v.at[k]], bufs[k], sems[k])

        def chunk(j, b, static):
            pltpu.make_async_copy(t_hbm.at[pl.ds(0, GCW)], bufs[b],
                                  sems[b]).wait()
            pb = (b - 1) % _R

            def drain():
                pltpu.make_async_copy(bufs[pb], out_hbm.at[pl.ds(base, GCW)],
                                      osems[pb]).wait()

            def prefetch():
                pltpu.async_copy(t_hbm.at[idx_v.at[j + _R - 1]], bufs[pb],
                                 sems[pb])

            if static:
                if j >= 1:
                    drain()
                if j + _R - 1 < nch:
                    prefetch()
            else:
                pl.when(j >= 1)(drain)
                pl.when(j + _R - 1 < nch)(prefetch)
            pltpu.async_copy(bufs[b], out_hbm.at[pl.ds(base + j * GCW, GCW)],
                             osems[b])

        def step(jj, carry):
            for b in range(_R):
                chunk(jj * _R + b, b, False)
            return carry

        lax.fori_loop(0, nch // _R, step, 0)
        for k in range(nch % _R):
            j = (nch // _R) * _R + k
            chunk(j, j % _R, True)
        lb = (nch - 1) % _R
        pltpu.make_async_copy(bufs[lb], out_hbm.at[pl.ds(base, GCW)],
                              osems[lb]).wait()

    def run(tflat, idx3):
        return pl.kernel(
            body,
            out_type=jax.ShapeDtypeStruct((NW * nch * GCW, D), jnp.float32),
            mesh=plsc.VectorSubcoreMesh(core_axis_name="c",
                                        subcore_axis_name="s"),
            scratch_types=(
                [pltpu.VMEM((nch, GCW), jnp.int32)]
                + [pltpu.VMEM((GCW, D), jnp.float32)] * _R
                + [pltpu.SemaphoreType.DMA] * (2 * _R)
            ),
        )(tflat, idx3)

    return run


def _make_scatter(nch):
    def body(e_hbm, idx_hbm, out_hbm, idx_v, zbuf, shared, *rest):
        bufs = rest[:_R]
        sems = rest[_R:2 * _R]
        asems = rest[2 * _R:3 * _R]
        cid = lax.axis_index("c")
        sid = lax.axis_index("s")
        wid = cid * NS + sid
        base = wid * (nch * SCW)
        pltpu.sync_copy(idx_hbm.at[wid], idx_v)
        for k in range(_R - 1):
            pltpu.async_copy(e_hbm.at[pl.ds(base + k * SCW, SCW)], bufs[k],
                             sems[k])

        def zrow(i, carry):
            r = i // (D // 16)
            c = (i % (D // 16)) * 16
            zbuf[r, pl.ds(c, 16)] = jnp.zeros((16,), jnp.float32)
            return carry

        lax.fori_loop(0, ZCW * (D // 16), zrow, 0)
        for k in range(NPC // ZCW):
            pltpu.sync_copy(zbuf, shared.at[pl.ds(sid * NPC + k * ZCW, ZCW)])
        plsc.subcore_barrier()

        def chunk(j, b, static):
            pltpu.make_async_copy(e_hbm.at[pl.ds(0, SCW)], bufs[b],
                                  sems[b]).wait()
            pltpu.async_copy(bufs[b], shared.at[idx_v.at[j]], asems[b],
                             add=True)
            pb = (b - 1) % _R

            def drain():
                pltpu.make_async_copy(bufs[pb], shared.at[pl.ds(0, SCW)],
                                      asems[pb]).wait()

            def prefetch():
                pltpu.async_copy(
                    e_hbm.at[pl.ds(base + (j + _R - 1) * SCW, SCW)],
                    bufs[pb], sems[pb])

            if static:
                if j >= 1:
                    drain()
                if j + _R - 1 < nch:
                    prefetch()
            else:
                pl.when(j >= 1)(drain)
                pl.when(j + _R - 1 < nch)(prefetch)

        def step(jj, carry):
            for b in range(_R):
                chunk(jj * _R + b, b, False)
            return carry

        lax.fori_loop(0, nch // _R, step, 0)
        for k in range(nch % _R):
            j = (nch // _R) * _R + k
            chunk(j, j % _R, True)
        lb = (nch - 1) % _R
        pltpu.make_async_copy(bufs[lb], shared.at[pl.ds(0, SCW)],
                              asems[lb]).wait()
        plsc.subcore_barrier()
        pltpu.sync_copy(shared.at[pl.ds(sid * NPC, NPC)],
                        out_hbm.at[cid, pl.ds(sid * NPC, NPC)])

    def run(e, idx3):
        return pl.kernel(
            body,
            out_type=jax.ShapeDtypeStruct((NC, NPAD, D), jnp.float32),
            mesh=plsc.VectorSubcoreMesh(core_axis_name="c",
                                        subcore_axis_name="s"),
            scratch_types=(
                [pltpu.VMEM((nch, SCW), jnp.int32),
                 pltpu.VMEM((ZCW, D), jnp.float32),
                 pltpu.VMEM_SHARED((NPAD, D), jnp.float32)]
                + [pltpu.VMEM((SCW, D), jnp.float32)] * _R
                + [pltpu.SemaphoreType.DMA] * (2 * _R)
            ),
        )(e, idx3)

    return run


_GATHERS = tuple(_make_gather(2 * h // NW // GCW) for h in HS)
_SCATTERS = tuple(_make_scatter(h // NW // SCW) for h in HS)


def _mm(a, w):
    return jnp.dot(a.astype(jnp.bfloat16), w.astype(jnp.bfloat16),
                   preferred_element_type=jnp.float32)


def _layer_norm(h, g, b):
    mu = jnp.mean(h, axis=-1, keepdims=True)
    var = jnp.mean((h - mu) ** 2, axis=-1, keepdims=True)
    return (h - mu) * lax.rsqrt(var + 1e-5) * g + b


def _pre_body(x_ref, wsd_ref, t_ref):
    x = x_ref[...]
    t_ref[0] = _mm(x, wsd_ref[0])
    t_ref[1] = _mm(x, wsd_ref[1])


def _preproj(x, wsd):
    nb = N // BN
    return pl.pallas_call(
        _pre_body,
        grid=(nb,),
        in_specs=[
            pl.BlockSpec((BN, D), lambda i: (i, 0)),
            pl.BlockSpec((2, D, D), lambda i: (0, 0, 0)),
        ],
        out_specs=pl.BlockSpec((2, BN, D), lambda i: (0, i, 0)),
        out_shape=jax.ShapeDtypeStruct((2, N, D), jnp.float32),
    )(x, wsd)


def _edge_body(e_ref, gs_ref, gd_ref, w1_ref, w2_ref, w3_ref,
               b1_ref, b2_ref, b3_ref, g_ref, bt_ref, o_ref):
    e = e_ref[...]
    h = _mm(e, w1_ref[...])
    h = h + gs_ref[...] + gd_ref[...]
    h = jnp.maximum(h + b1_ref[...], 0.0)
    h = jnp.maximum(_mm(h, w2_ref[...]) + b2_ref[...], 0.0)
    h = _mm(h, w3_ref[...]) + b3_ref[...]
    o_ref[...] = _layer_norm(h, g_ref[...], bt_ref[...]) + e


def _edge_mlp(e, G, w1e, w2, w3, b1, b2, b3, g, bt):
    nr = e.shape[0]
    nb = nr // BE
    vspec = pl.BlockSpec((1, D), lambda i: (0, 0))
    wspec = pl.BlockSpec((D, D), lambda i: (0, 0))
    return pl.pallas_call(
        _edge_body,
        grid=(nb,),
        in_specs=[
            pl.BlockSpec((BE, D), lambda i: (i, 0)),
            pl.BlockSpec((BE, D), lambda i: (i, 0)),
            pl.BlockSpec((BE, D), lambda i: (i + nb, 0)),
            wspec, wspec, wspec, vspec, vspec, vspec, vspec, vspec,
        ],
        out_specs=pl.BlockSpec((BE, D), lambda i: (i, 0)),
        out_shape=jax.ShapeDtypeStruct((nr, D), jnp.float32),
    )(e, G, G, w1e, w2, w3, b1, b2, b3, g, bt)


def _node_body(*refs):
    aggs = refs[:len(HS)]
    (x_ref, w1a_ref, w1x_ref, w2_ref, w3_ref,
     b1_ref, b2_ref, b3_ref, g_ref, bt_ref, *rest) = refs[len(HS):]
    x = x_ref[...]
    a = aggs[0][0] + aggs[0][1]
    for ar in aggs[1:]:
        a = a + ar[0] + ar[1]
    h = _mm(a, w1a_ref[...]) + _mm(x, w1x_ref[...])
    h = jnp.maximum(h + b1_ref[...], 0.0)
    h = jnp.maximum(_mm(h, w2_ref[...]) + b2_ref[...], 0.0)
    h = _mm(h, w3_ref[...]) + b3_ref[...]
    xn = _layer_norm(h, g_ref[...], bt_ref[...]) + x
    if len(rest) == 1:
        rest[0][...] = xn
    else:
        wsd_ref, xo_ref, t_ref = rest
        xo_ref[...] = xn
        t_ref[0] = _mm(xn, wsd_ref[0])
        t_ref[1] = _mm(xn, wsd_ref[1])


def _node_mlp(aggs, x, w1a, w1x, w2, w3, b1, b2, b3, g, bt, wsd_next):
    nb = N // BN
    vspec = pl.BlockSpec((1, D), lambda i: (0, 0))
    wspec = pl.BlockSpec((D, D), lambda i: (0, 0))
    aspec = pl.BlockSpec((NC, BN, D), lambda i: (0, i, 0))
    in_specs = [aspec] * len(HS) + [
        pl.BlockSpec((BN, D), lambda i: (i, 0)),
        wspec, wspec, wspec, wspec, vspec, vspec, vspec, vspec, vspec,
    ]
    xspec = pl.BlockSpec((BN, D), lambda i: (i, 0))
    xshape = jax.ShapeDtypeStruct((N, D), jnp.float32)
    args = list(aggs) + [x, w1a, w1x, w2, w3, b1, b2, b3, g, bt]
    if wsd_next is None:
        return pl.pallas_call(
            _node_body, grid=(nb,), in_specs=in_specs,
            out_specs=xspec, out_shape=xshape,
        )(*args)
    in_specs.append(pl.BlockSpec((2, D, D), lambda i: (0, 0, 0)))
    return pl.pallas_call(
        _node_body, grid=(nb,), in_specs=in_specs,
        out_specs=(xspec, pl.BlockSpec((2, BN, D), lambda i: (0, i, 0))),
        out_shape=(xshape, jax.ShapeDtypeStruct((2, N, D), jnp.float32)),
    )(*args, wsd_next)


def kernel(node_features, edge_features, edge_index, params):
    src = edge_index[0]
    dst = edge_index[1]
    e_parts = []
    idx_g = []
    idx_s = []
    off = 0
    for h in HS:
        e_parts.append(lax.dynamic_slice(edge_features, (off, 0), (h, D)))
        s_h = lax.dynamic_slice(src, (off,), (h,))
        d_h = lax.dynamic_slice(dst, (off,), (h,))
        idx_g.append(jnp.concatenate([s_h, d_h + N])
                     .reshape(NW, 2 * h // NW // GCW, GCW))
        idx_s.append(d_h.reshape(NW, h // NW // SCW, SCW))
        off += h
    e_parts = tuple(e_parts)

    def wsd(l):
        w1 = params['e%d_W1' % l]
        return jnp.stack([w1[D:2 * D], w1[2 * D:3 * D]])

    def v(pre, s):
        return params[pre + s].reshape(1, D)

    x = node_features
    T = _preproj(x, wsd(0))
    for l in range(L):
        ep = 'e%d' % l
        np_ = 'n%d' % l
        tflat = T.reshape(2 * N, D)
        new_parts = []
        aggs = []
        for h in range(len(HS)):
            G = _GATHERS[h](tflat, idx_g[h])
            eh = _edge_mlp(e_parts[h], G, params[ep + '_W1'][:D],
                           params[ep + '_W2'], params[ep + '_W3'],
                           v(ep, '_b1'), v(ep, '_b2'), v(ep, '_b3'),
                           v(ep, '_g'), v(ep, '_beta'))
            new_parts.append(eh)
            aggs.append(_SCATTERS[h](eh, idx_s[h]))
        e_parts = tuple(new_parts)
        out = _node_mlp(aggs, x, params[np_ + '_W1'][:D],
                        params[np_ + '_W1'][D:], params[np_ + '_W2'],
                        params[np_ + '_W3'], v(np_, '_b1'), v(np_, '_b2'),
                        v(np_, '_b3'), v(np_, '_g'), v(np_, '_beta'),
                        wsd(l + 1) if l < L - 1 else None)
        if l < L - 1:
            x, T = out
        else:
            x = out
    return x
